# Initial kernel scaffold; baseline (speedup 1.0000x reference)
import jax
import jax.numpy as jnp
from jax.experimental import pallas as pl


def kernel(x, edge_index, W1, a1s, a1d, b1, W3, a3s, a3d, b3, p):
    raise NotImplementedError("write your pallas kernel here")


# speedup vs baseline: 1.1208x; 1.1208x over previous
import jax
import jax.numpy as jnp
import numpy as np
from jax.experimental import pallas as pl


def _mm_kernel(x_ref, w_ref, o_ref):
    o_ref[...] = jnp.dot(x_ref[...], w_ref[...],
                         preferred_element_type=jnp.float32)


def _pallas_mm(x, W):
    N, K = x.shape
    Ko, F = W.shape
    BR = 1000
    return pl.pallas_call(
        _mm_kernel,
        grid=(N // BR,),
        in_specs=[pl.BlockSpec((BR, K), lambda i: (i, 0)),
                  pl.BlockSpec((K, F), lambda i: (0, 0))],
        out_specs=pl.BlockSpec((BR, F), lambda i: (i, 0)),
        out_shape=jax.ShapeDtypeStruct((N, F), jnp.float32),
    )(x, W)


def _gat(x, edge_index, W, a_s, a_d, b):
    N = x.shape[0]
    loop = jnp.arange(N, dtype=edge_index.dtype)
    src = jnp.concatenate([edge_index[0], loop])
    dst = jnp.concatenate([edge_index[1], loop])
    h = _pallas_mm(x, W)
    alpha_src = h @ a_s
    alpha_dst = h @ a_d
    e = jax.nn.leaky_relu(alpha_src[src] + alpha_dst[dst], negative_slope=0.2)
    emax = jax.ops.segment_max(e, dst, num_segments=N)
    emax = jax.lax.stop_gradient(emax)
    ee = jnp.exp(e - emax[dst])
    den = jax.ops.segment_sum(ee, dst, num_segments=N)
    alpha = ee / (den[dst] + 1e-16)
    out = jax.ops.segment_sum(h[src] * alpha[:, None], dst, num_segments=N)
    return out + b


def _l2norm(x):
    n = jnp.linalg.norm(x, axis=1, keepdims=True)
    return x / jnp.maximum(n, 1e-12)


def kernel(x, edge_index, W1, a1s, a1d, b1, W3, a3s, a3d, b3, p):
    h = jax.nn.relu(_gat(x, edge_index, W1, a1s, a1d, b1))
    h = _l2norm(h)
    h = _gat(h, edge_index, W3, a3s, a3d, b3)
    h = _l2norm(h)
    score = jnp.tanh((h @ p) / jnp.maximum(jnp.linalg.norm(p), 1e-12))
    k = int(np.ceil(0.5 * h.shape[0]))
    vals, perm = jax.lax.top_k(score, k)
    return h[perm] * vals[:, None]


# speedup vs baseline: 2.5724x; 2.2951x over previous
import functools

import jax
import jax.numpy as jnp
import numpy as np
from jax import lax
from jax.experimental import pallas as pl
from jax.experimental.pallas import tpu as pltpu
from jax.experimental.pallas import tpu_sc as plsc

N = 10000
NP = 10240
EFULL = 330000
EPG = 360448
PER_W = EPG // 32
NCHUNK = PER_W // 128
NVREG = PER_W // 16

_MESH = plsc.VectorSubcoreMesh(core_axis_name="c", subcore_axis_name="s")
_SC_PARAMS = pltpu.CompilerParams(needs_layout_passes=False)


def _mm_kernel(x_ref, w_ref, o_ref):
    o_ref[...] = jnp.dot(x_ref[...], w_ref[...],
                         preferred_element_type=jnp.float32)


def _pallas_mm(x, W):
    Nr, K = x.shape
    _, F = W.shape
    BR = 1000
    return pl.pallas_call(
        _mm_kernel,
        grid=(Nr // BR,),
        in_specs=[pl.BlockSpec((BR, K), lambda i: (i, 0)),
                  pl.BlockSpec((K, F), lambda i: (0, 0))],
        out_specs=pl.BlockSpec((BR, F), lambda i: (i, 0)),
        out_shape=jax.ShapeDtypeStruct((Nr, F), jnp.float32),
    )(x, W)


def _pallas_proj(h, a):
    Nr, F = h.shape
    BR = 1000
    out = pl.pallas_call(
        _mm_kernel,
        grid=(Nr // BR,),
        in_specs=[pl.BlockSpec((BR, F), lambda i: (i, 0)),
                  pl.BlockSpec((F, 1), lambda i: (0, 0))],
        out_specs=pl.BlockSpec((BR, 1), lambda i: (i, 0)),
        out_shape=jax.ShapeDtypeStruct((Nr, 1), jnp.float32),
    )(h, a.reshape(F, 1))
    return out.reshape(Nr)


def _ee_kernel(e_ref, m_ref, o_ref):
    o_ref[...] = jnp.exp(e_ref[...] - m_ref[...])


def _alpha_kernel(ee_ref, d_ref, o_ref):
    o_ref[...] = ee_ref[...] / (d_ref[...] + 1e-16)


def _ew2d(kernel_fn, a, b):
    a2 = a.reshape(2816, 128)
    b2 = b.reshape(2816, 128)
    out = pl.pallas_call(
        kernel_fn,
        grid=(4,),
        in_specs=[pl.BlockSpec((704, 128), lambda i: (i, 0)),
                  pl.BlockSpec((704, 128), lambda i: (i, 0))],
        out_specs=pl.BlockSpec((704, 128), lambda i: (i, 0)),
        out_shape=jax.ShapeDtypeStruct((2816, 128), jnp.float32),
    )(a2, b2)
    return out.reshape(EPG)


def _make_upd_kernel(D):
    def _upd_kernel(g_ref, al_ref, o_ref):
        o_ref[...] = g_ref[:, :D] * al_ref[...]
    return _upd_kernel


def _pallas_upd(g, alpha, D):
    Er = g.shape[0]
    BR = 4096
    return pl.pallas_call(
        _make_upd_kernel(D),
        grid=(Er // BR,),
        in_specs=[pl.BlockSpec((BR, 128), lambda i: (i, 0)),
                  pl.BlockSpec((BR, 1), lambda i: (i, 0))],
        out_specs=pl.BlockSpec((BR, D), lambda i: (i, 0)),
        out_shape=jax.ShapeDtypeStruct((Er, D), jnp.float32),
    )(g, alpha.reshape(Er, 1))


def _relu_b_kernel(x_ref, b_ref, o_ref):
    o_ref[...] = jax.nn.relu(x_ref[...] + b_ref[...])


def _pallas_relu_b(x, b):
    Nr, D = x.shape
    BR = 1000
    return pl.pallas_call(
        _relu_b_kernel,
        grid=(Nr // BR,),
        in_specs=[pl.BlockSpec((BR, D), lambda i: (i, 0)),
                  pl.BlockSpec((1, D), lambda i: (0, 0))],
        out_specs=pl.BlockSpec((BR, D), lambda i: (i, 0)),
        out_shape=jax.ShapeDtypeStruct((Nr, D), jnp.float32),
    )(x, b.reshape(1, D))


def _addb_kernel(x_ref, b_ref, o_ref):
    o_ref[...] = x_ref[...] + b_ref[...]


def _pallas_addb(x, b):
    Nr, D = x.shape
    BR = 1000
    return pl.pallas_call(
        _addb_kernel,
        grid=(Nr // BR,),
        in_specs=[pl.BlockSpec((BR, D), lambda i: (i, 0)),
                  pl.BlockSpec((1, D), lambda i: (0, 0))],
        out_specs=pl.BlockSpec((BR, D), lambda i: (i, 0)),
        out_shape=jax.ShapeDtypeStruct((Nr, D), jnp.float32),
    )(x, b.reshape(1, D))


def _score_kernel(h_ref, p_ref, o_ref):
    p = p_ref[...]
    pn = jnp.sqrt(jnp.sum(p * p))
    u = jnp.dot(h_ref[...], p.reshape(-1, 1),
                preferred_element_type=jnp.float32)
    o_ref[...] = jnp.tanh(u / jnp.maximum(pn, 1e-12))


def _pallas_score(h, p):
    Nr, F = h.shape
    BR = 1000
    out = pl.pallas_call(
        _score_kernel,
        grid=(Nr // BR,),
        in_specs=[pl.BlockSpec((BR, F), lambda i: (i, 0)),
                  pl.BlockSpec((1, F), lambda i: (0, 0))],
        out_specs=pl.BlockSpec((BR, 1), lambda i: (i, 0)),
        out_shape=jax.ShapeDtypeStruct((Nr, 1), jnp.float32),
    )(h, p.reshape(1, F))
    return out.reshape(Nr)


def _rank_kernel(sc_ref, sr_ref, o_ref):
    pid = pl.program_id(0)
    si = sc_ref[...]
    gi = jax.lax.broadcasted_iota(jnp.int32, (1024, 1), 0) + pid * 1024
    acc = jnp.zeros((1024, 1), jnp.int32)
    for c in range(8):
        sj = sr_ref[:, c * 1280:(c + 1) * 1280]
        gj = jax.lax.broadcasted_iota(jnp.int32, (1, 1280), 1) + c * 1280
        hit = (sj > si) | ((sj == si) & (gj < gi))
        acc = acc + jnp.sum(hit.astype(jnp.int32), axis=1, keepdims=True)
    o_ref[...] = acc


def _pallas_rank(score_p):
    out = pl.pallas_call(
        _rank_kernel,
        grid=(NP // 1024,),
        in_specs=[pl.BlockSpec((1024, 1), lambda i: (i, 0)),
                  pl.BlockSpec((1, NP), lambda i: (0, 0))],
        out_specs=pl.BlockSpec((1024, 1), lambda i: (i, 0)),
        out_shape=jax.ShapeDtypeStruct((NP, 1), jnp.int32),
    )(score_p.reshape(NP, 1), score_p.reshape(1, NP))
    return out.reshape(NP)


@functools.partial(
    pl.kernel,
    out_type=jax.ShapeDtypeStruct((EPG,), jnp.float32),
    mesh=_MESH,
    compiler_params=_SC_PARAMS,
    scratch_types=[
        pltpu.VMEM((NP,), jnp.float32),
        pltpu.VMEM((NP,), jnp.float32),
        pltpu.VMEM((PER_W,), jnp.int32),
        pltpu.VMEM((PER_W,), jnp.int32),
        pltpu.VMEM((PER_W,), jnp.float32),
    ],
)
def _sc_edge_logits(src_hbm, dst_hbm, as_hbm, ad_hbm, e_hbm,
                    as_v, ad_v, sv, dv, ev):
    wid = lax.axis_index("c") * 16 + lax.axis_index("s")
    base = wid * PER_W
    pltpu.sync_copy(as_hbm, as_v)
    pltpu.sync_copy(ad_hbm, ad_v)
    pltpu.sync_copy(src_hbm.at[pl.ds(base, PER_W)], sv)
    pltpu.sync_copy(dst_hbm.at[pl.ds(base, PER_W)], dv)

    def body(i, _):
        s16 = sv[pl.ds(i * 16, 16)]
        d16 = dv[pl.ds(i * 16, 16)]
        a = plsc.load_gather(as_v, [s16])
        bvals = plsc.load_gather(ad_v, [d16])
        v = a + bvals
        ev[pl.ds(i * 16, 16)] = jnp.where(v >= 0, v, jnp.float32(0.2) * v)
        return ()
    lax.fori_loop(0, NVREG, body, ())
    pltpu.sync_copy(ev, e_hbm.at[pl.ds(base, PER_W)])


@functools.partial(
    pl.kernel,
    out_type=jax.ShapeDtypeStruct((EPG,), jnp.float32),
    mesh=_MESH,
    compiler_params=_SC_PARAMS,
    scratch_types=[
        pltpu.VMEM((NP,), jnp.float32),
        pltpu.VMEM((PER_W,), jnp.int32),
        pltpu.VMEM((PER_W,), jnp.float32),
    ],
)
def _sc_gather1(table_hbm, idx_hbm, o_hbm, tv, iv, ov):
    wid = lax.axis_index("c") * 16 + lax.axis_index("s")
    base = wid * PER_W
    pltpu.sync_copy(table_hbm, tv)
    pltpu.sync_copy(idx_hbm.at[pl.ds(base, PER_W)], iv)

    def body(i, _):
        i16 = iv[pl.ds(i * 16, 16)]
        ov[pl.ds(i * 16, 16)] = plsc.load_gather(tv, [i16])
        return ()
    lax.fori_loop(0, NVREG, body, ())
    pltpu.sync_copy(ov, o_hbm.at[pl.ds(base, PER_W)])


@functools.partial(
    pl.kernel,
    out_type=jax.ShapeDtypeStruct((EPG, 128), jnp.float32),
    mesh=_MESH,
    compiler_params=_SC_PARAMS,
    scratch_types=[
        pltpu.VMEM((PER_W,), jnp.int32),
        pltpu.VMEM((128, 128), jnp.float32),
        pltpu.SemaphoreType.DMA,
    ],
)
def _sc_rowgather(h_hbm, src_hbm, o_hbm, iv, rows_v, sem):
    wid = lax.axis_index("c") * 16 + lax.axis_index("s")
    pltpu.sync_copy(src_hbm.at[pl.ds(wid * PER_W, PER_W)], iv)

    def body(r, _):
        pltpu.async_copy(
            h_hbm.at[iv.at[pl.ds(r * 128, 128)]], rows_v, sem).wait()
        pltpu.sync_copy(
            rows_v, o_hbm.at[pl.ds(wid * PER_W + r * 128, 128), :])
        return ()
    lax.fori_loop(0, NCHUNK, body, ())


@functools.partial(
    pl.kernel,
    out_type=jax.ShapeDtypeStruct((NP, 128), jnp.float32),
    mesh=_MESH,
    compiler_params=_SC_PARAMS,
    scratch_types=[
        pltpu.VMEM((NP,), jnp.float32),
        pltpu.VMEM((80, 128), jnp.int32),
        pltpu.VMEM((640, 128), jnp.float32),
        pltpu.SemaphoreType.DMA,
    ],
)
def _sc_final(hf_hbm, score_hbm, rank2d_hbm, o_hbm, sv, rv2, rows_v, sem):
    wid = lax.axis_index("c") * 16 + lax.axis_index("s")

    @pl.when(wid < 16)
    def _():
        base = wid * 640
        pltpu.sync_copy(score_hbm, sv)
        pltpu.sync_copy(rank2d_hbm, rv2)
        pltpu.sync_copy(hf_hbm.at[pl.ds(base, 640), :], rows_v)

        def scale_body(r, _):
            s16 = plsc.load_gather(sv, [jnp.full((16,), base + r, jnp.int32)])
            for k in range(8):
                rows_v[r, pl.ds(k * 16, 16)] = (
                    rows_v[r, pl.ds(k * 16, 16)] * s16)
            return ()
        lax.fori_loop(0, 640, scale_body, ())

        def scat_body(j, _):
            pltpu.async_copy(
                rows_v.at[pl.ds(j * 128, 128), :],
                o_hbm.at[rv2.at[wid * 5 + j]], sem).wait()
            return ()
        lax.fori_loop(0, 5, scat_body, ())


def _pad1(v, n, val=0.0):
    return jnp.pad(v, (0, n - v.shape[0]), constant_values=val)


def _gat_fast(x, W, a_s, a_d, src_p, dst_p, dst, D):
    Nn = x.shape[0]
    h = _pallas_mm(x, W)
    alpha_src = _pallas_proj(h, a_s)
    alpha_dst = _pallas_proj(h, a_d)
    e = _sc_edge_logits(src_p, dst_p, _pad1(alpha_src, NP),
                        _pad1(alpha_dst, NP))
    emax = jax.ops.segment_max(e[:EFULL], dst, num_segments=Nn)
    emax = jax.lax.stop_gradient(emax)
    emd = _sc_gather1(_pad1(emax, NP), dst_p)
    ee = _ew2d(_ee_kernel, e, emd)
    den = jax.ops.segment_sum(ee[:EFULL], dst, num_segments=Nn)
    dend = _sc_gather1(_pad1(den, NP), dst_p)
    alpha = _ew2d(_alpha_kernel, ee, dend)
    h_p = jnp.pad(h, ((0, NP - Nn), (0, 128 - D)))
    g = _sc_rowgather(h_p, src_p)
    upd = _pallas_upd(g, alpha, D)
    out = jax.ops.segment_sum(upd[:EFULL], dst, num_segments=Nn)
    return out, h


def _l2norm(x):
    n = jnp.linalg.norm(x, axis=1, keepdims=True)
    return x / jnp.maximum(n, 1e-12)


def kernel(x, edge_index, W1, a1s, a1d, b1, W3, a3s, a3d, b3, p):
    loop = jnp.arange(N, dtype=jnp.int32)
    src = jnp.concatenate([edge_index[0], loop])
    dst = jnp.concatenate([edge_index[1], loop])
    src_p = _pad1(src, EPG, N)
    dst_p = _pad1(dst, EPG, N)

    o1, _ = _gat_fast(x, W1, a1s, a1d, src_p, dst_p, dst, 16)
    h = _pallas_relu_b(o1, b1)
    h = _l2norm(h)
    o2, _ = _gat_fast(h, W3, a3s, a3d, src_p, dst_p, dst, 128)
    h2 = _l2norm(_pallas_addb(o2, b3))
    score = _pallas_score(h2, p)

    score_pad = _pad1(score, NP, -3e38)
    rank = _pallas_rank(score_pad)
    hf_p = jnp.pad(h2, ((0, NP - N), (0, 0)))
    return _sc_final(hf_p, score_pad, rank.reshape(80, 128))[:5000]


# speedup vs baseline: 3.7086x; 1.4417x over previous
import functools

import jax
import jax.numpy as jnp
import numpy as np
from jax import lax
from jax.experimental import pallas as pl
from jax.experimental.pallas import tpu as pltpu
from jax.experimental.pallas import tpu_sc as plsc

N = 10000
NP = 10240
EFULL = 330000
EPG = 360448
PER_W = EPG // 32
NCHUNK = PER_W // 128
NVREG = PER_W // 16

_MESH = plsc.VectorSubcoreMesh(core_axis_name="c", subcore_axis_name="s")
_SC_PARAMS = pltpu.CompilerParams(needs_layout_passes=False)


def _mm_kernel(x_ref, w_ref, o_ref):
    o_ref[...] = jnp.dot(x_ref[...], w_ref[...],
                         preferred_element_type=jnp.float32)


def _pallas_mm(x, W):
    Nr, K = x.shape
    _, F = W.shape
    BR = 1000
    return pl.pallas_call(
        _mm_kernel,
        grid=(Nr // BR,),
        in_specs=[pl.BlockSpec((BR, K), lambda i: (i, 0)),
                  pl.BlockSpec((K, F), lambda i: (0, 0))],
        out_specs=pl.BlockSpec((BR, F), lambda i: (i, 0)),
        out_shape=jax.ShapeDtypeStruct((Nr, F), jnp.float32),
    )(x, W)


def _pallas_proj(h, a):
    Nr, F = h.shape
    BR = 1000
    out = pl.pallas_call(
        _mm_kernel,
        grid=(Nr // BR,),
        in_specs=[pl.BlockSpec((BR, F), lambda i: (i, 0)),
                  pl.BlockSpec((F, 1), lambda i: (0, 0))],
        out_specs=pl.BlockSpec((BR, 1), lambda i: (i, 0)),
        out_shape=jax.ShapeDtypeStruct((Nr, 1), jnp.float32),
    )(h, a.reshape(F, 1))
    return out.reshape(Nr)


def _ee_kernel(e_ref, m_ref, o_ref):
    o_ref[...] = jnp.exp(e_ref[...] - m_ref[...])


def _alpha_kernel(ee_ref, d_ref, o_ref):
    o_ref[...] = ee_ref[...] / (d_ref[...] + 1e-16)


def _ew2d(kernel_fn, a, b):
    a2 = a.reshape(2816, 128)
    b2 = b.reshape(2816, 128)
    out = pl.pallas_call(
        kernel_fn,
        grid=(4,),
        in_specs=[pl.BlockSpec((704, 128), lambda i: (i, 0)),
                  pl.BlockSpec((704, 128), lambda i: (i, 0))],
        out_specs=pl.BlockSpec((704, 128), lambda i: (i, 0)),
        out_shape=jax.ShapeDtypeStruct((2816, 128), jnp.float32),
    )(a2, b2)
    return out.reshape(EPG)


def _make_upd_kernel(D):
    def _upd_kernel(g_ref, al_ref, o_ref):
        o_ref[...] = g_ref[:, :D] * al_ref[...]
    return _upd_kernel


def _pallas_upd(g, alpha, D):
    Er = g.shape[0]
    BR = 4096
    return pl.pallas_call(
        _make_upd_kernel(D),
        grid=(Er // BR,),
        in_specs=[pl.BlockSpec((BR, 128), lambda i: (i, 0)),
                  pl.BlockSpec((BR, 1), lambda i: (i, 0))],
        out_specs=pl.BlockSpec((BR, D), lambda i: (i, 0)),
        out_shape=jax.ShapeDtypeStruct((Er, D), jnp.float32),
    )(g, alpha.reshape(Er, 1))


def _relu_b_kernel(x_ref, b_ref, o_ref):
    o_ref[...] = jax.nn.relu(x_ref[...] + b_ref[...])


def _pallas_relu_b(x, b):
    Nr, D = x.shape
    BR = 1000
    return pl.pallas_call(
        _relu_b_kernel,
        grid=(Nr // BR,),
        in_specs=[pl.BlockSpec((BR, D), lambda i: (i, 0)),
                  pl.BlockSpec((1, D), lambda i: (0, 0))],
        out_specs=pl.BlockSpec((BR, D), lambda i: (i, 0)),
        out_shape=jax.ShapeDtypeStruct((Nr, D), jnp.float32),
    )(x, b.reshape(1, D))


def _addb_kernel(x_ref, b_ref, o_ref):
    o_ref[...] = x_ref[...] + b_ref[...]


def _pallas_addb(x, b):
    Nr, D = x.shape
    BR = 1000
    return pl.pallas_call(
        _addb_kernel,
        grid=(Nr // BR,),
        in_specs=[pl.BlockSpec((BR, D), lambda i: (i, 0)),
                  pl.BlockSpec((1, D), lambda i: (0, 0))],
        out_specs=pl.BlockSpec((BR, D), lambda i: (i, 0)),
        out_shape=jax.ShapeDtypeStruct((Nr, D), jnp.float32),
    )(x, b.reshape(1, D))


def _score_kernel(h_ref, p_ref, o_ref):
    p = p_ref[...]
    pn = jnp.sqrt(jnp.sum(p * p))
    u = jnp.dot(h_ref[...], p.reshape(-1, 1),
                preferred_element_type=jnp.float32)
    o_ref[...] = jnp.tanh(u / jnp.maximum(pn, 1e-12))


def _pallas_score(h, p):
    Nr, F = h.shape
    BR = 1000
    out = pl.pallas_call(
        _score_kernel,
        grid=(Nr // BR,),
        in_specs=[pl.BlockSpec((BR, F), lambda i: (i, 0)),
                  pl.BlockSpec((1, F), lambda i: (0, 0))],
        out_specs=pl.BlockSpec((BR, 1), lambda i: (i, 0)),
        out_shape=jax.ShapeDtypeStruct((Nr, 1), jnp.float32),
    )(h, p.reshape(1, F))
    return out.reshape(Nr)


def _rank_kernel(sc_ref, sr_ref, o_ref):
    pid = pl.program_id(0)
    si = sc_ref[...]
    gi = jax.lax.broadcasted_iota(jnp.int32, (1024, 1), 0) + pid * 1024
    acc = jnp.zeros((1024, 1), jnp.int32)
    for c in range(8):
        sj = sr_ref[:, c * 1280:(c + 1) * 1280]
        gj = jax.lax.broadcasted_iota(jnp.int32, (1, 1280), 1) + c * 1280
        hit = (sj > si) | ((sj == si) & (gj < gi))
        acc = acc + jnp.sum(hit.astype(jnp.int32), axis=1, keepdims=True)
    o_ref[...] = acc


def _pallas_rank(score_p):
    out = pl.pallas_call(
        _rank_kernel,
        grid=(NP // 1024,),
        in_specs=[pl.BlockSpec((1024, 1), lambda i: (i, 0)),
                  pl.BlockSpec((1, NP), lambda i: (0, 0))],
        out_specs=pl.BlockSpec((1024, 1), lambda i: (i, 0)),
        out_shape=jax.ShapeDtypeStruct((NP, 1), jnp.int32),
    )(score_p.reshape(NP, 1), score_p.reshape(1, NP))
    return out.reshape(NP)


@functools.partial(
    pl.kernel,
    out_type=jax.ShapeDtypeStruct((EPG,), jnp.float32),
    mesh=_MESH,
    compiler_params=_SC_PARAMS,
    scratch_types=[
        pltpu.VMEM((NP,), jnp.float32),
        pltpu.VMEM((NP,), jnp.float32),
        pltpu.VMEM((PER_W,), jnp.int32),
        pltpu.VMEM((PER_W,), jnp.int32),
        pltpu.VMEM((PER_W,), jnp.float32),
    ],
)
def _sc_edge_logits(src_hbm, dst_hbm, as_hbm, ad_hbm, e_hbm,
                    as_v, ad_v, sv, dv, ev):
    wid = lax.axis_index("c") * 16 + lax.axis_index("s")
    base = wid * PER_W
    pltpu.sync_copy(as_hbm, as_v)
    pltpu.sync_copy(ad_hbm, ad_v)
    pltpu.sync_copy(src_hbm.at[pl.ds(base, PER_W)], sv)
    pltpu.sync_copy(dst_hbm.at[pl.ds(base, PER_W)], dv)

    def body(i, _):
        s16 = sv[pl.ds(i * 16, 16)]
        d16 = dv[pl.ds(i * 16, 16)]
        a = plsc.load_gather(as_v, [s16])
        bvals = plsc.load_gather(ad_v, [d16])
        v = a + bvals
        ev[pl.ds(i * 16, 16)] = jnp.where(v >= 0, v, jnp.float32(0.2) * v)
        return ()
    lax.fori_loop(0, NVREG, body, ())
    pltpu.sync_copy(ev, e_hbm.at[pl.ds(base, PER_W)])


@functools.partial(
    pl.kernel,
    out_type=jax.ShapeDtypeStruct((EPG,), jnp.float32),
    mesh=_MESH,
    compiler_params=_SC_PARAMS,
    scratch_types=[
        pltpu.VMEM((NP,), jnp.float32),
        pltpu.VMEM((PER_W,), jnp.int32),
        pltpu.VMEM((PER_W,), jnp.float32),
    ],
)
def _sc_gather1(table_hbm, idx_hbm, o_hbm, tv, iv, ov):
    wid = lax.axis_index("c") * 16 + lax.axis_index("s")
    base = wid * PER_W
    pltpu.sync_copy(table_hbm, tv)
    pltpu.sync_copy(idx_hbm.at[pl.ds(base, PER_W)], iv)

    def body(i, _):
        i16 = iv[pl.ds(i * 16, 16)]
        ov[pl.ds(i * 16, 16)] = plsc.load_gather(tv, [i16])
        return ()
    lax.fori_loop(0, NVREG, body, ())
    pltpu.sync_copy(ov, o_hbm.at[pl.ds(base, PER_W)])


@functools.partial(
    pl.kernel,
    out_type=jax.ShapeDtypeStruct((EPG, 128), jnp.float32),
    mesh=_MESH,
    compiler_params=_SC_PARAMS,
    scratch_types=[
        pltpu.VMEM((PER_W,), jnp.int32),
        pltpu.VMEM((128, 128), jnp.float32),
        pltpu.VMEM((128, 128), jnp.float32),
        pltpu.SemaphoreType.DMA,
        pltpu.SemaphoreType.DMA,
    ],
)
def _sc_rowgather(h_hbm, src_hbm, o_hbm, iv, rows_a, rows_b, sem_a, sem_b):
    wid = lax.axis_index("c") * 16 + lax.axis_index("s")
    base = wid * PER_W
    pltpu.sync_copy(src_hbm.at[pl.ds(base, PER_W)], iv)

    pltpu.async_copy(h_hbm.at[iv.at[pl.ds(0, 128)]], rows_a, sem_a)

    def body(j, _):
        c0 = 2 * j
        c1 = 2 * j + 1
        pltpu.async_copy(
            h_hbm.at[iv.at[pl.ds(c1 * 128, 128)]], rows_b, sem_b)
        pltpu.make_async_copy(
            h_hbm.at[iv.at[pl.ds(0, 128)]], rows_a, sem_a).wait()
        pltpu.sync_copy(rows_a, o_hbm.at[pl.ds(base + c0 * 128, 128), :])

        @pl.when(j + 1 < NCHUNK // 2)
        def _():
            pltpu.async_copy(
                h_hbm.at[iv.at[pl.ds((c0 + 2) * 128, 128)]], rows_a, sem_a)
        pltpu.make_async_copy(
            h_hbm.at[iv.at[pl.ds(0, 128)]], rows_b, sem_b).wait()
        pltpu.sync_copy(rows_b, o_hbm.at[pl.ds(base + c1 * 128, 128), :])
        return ()
    lax.fori_loop(0, NCHUNK // 2, body, ())


@functools.partial(
    pl.kernel,
    out_type=jax.ShapeDtypeStruct((NP, 128), jnp.float32),
    mesh=_MESH,
    compiler_params=_SC_PARAMS,
    scratch_types=[
        pltpu.VMEM((NP,), jnp.float32),
        pltpu.VMEM((80, 128), jnp.int32),
        pltpu.VMEM((640, 128), jnp.float32),
        pltpu.SemaphoreType.DMA,
    ],
)
def _sc_final(hf_hbm, score_hbm, rank2d_hbm, o_hbm, sv, rv2, rows_v, sem):
    wid = lax.axis_index("c") * 16 + lax.axis_index("s")

    @pl.when(wid < 16)
    def _():
        base = wid * 640
        pltpu.sync_copy(score_hbm, sv)
        pltpu.sync_copy(rank2d_hbm, rv2)
        pltpu.sync_copy(hf_hbm.at[pl.ds(base, 640), :], rows_v)

        def scale_body(r, _):
            s16 = plsc.load_gather(sv, [jnp.full((16,), base + r, jnp.int32)])
            for k in range(8):
                rows_v[r, pl.ds(k * 16, 16)] = (
                    rows_v[r, pl.ds(k * 16, 16)] * s16)
            return ()
        lax.fori_loop(0, 640, scale_body, ())

        def scat_body(j, _):
            pltpu.async_copy(
                rows_v.at[pl.ds(j * 128, 128), :],
                o_hbm.at[rv2.at[wid * 5 + j]], sem).wait()
            return ()
        lax.fori_loop(0, 5, scat_body, ())


def _pad1(v, n, val=0.0):
    return jnp.pad(v, (0, n - v.shape[0]), constant_values=val)


def _gat_fast(x, W, a_s, a_d, src_p, dst_p, dst, D):
    Nn = x.shape[0]
    h = _pallas_mm(x, W)
    alpha_src = _pallas_proj(h, a_s)
    alpha_dst = _pallas_proj(h, a_d)
    e = _sc_edge_logits(src_p, dst_p, _pad1(alpha_src, NP),
                        _pad1(alpha_dst, NP))
    emax = jax.ops.segment_max(e[:EFULL], dst, num_segments=Nn)
    emax = jax.lax.stop_gradient(emax)
    emd = _sc_gather1(_pad1(emax, NP), dst_p)
    ee = _ew2d(_ee_kernel, e, emd)
    den = jax.ops.segment_sum(ee[:EFULL], dst, num_segments=Nn)
    dend = _sc_gather1(_pad1(den, NP), dst_p)
    alpha = _ew2d(_alpha_kernel, ee, dend)
    h_p = jnp.pad(h, ((0, NP - Nn), (0, 128 - D)))
    g = _sc_rowgather(h_p, src_p)
    upd = _pallas_upd(g, alpha, D)
    out = jax.ops.segment_sum(upd[:EFULL], dst, num_segments=Nn)
    return out, h


def _l2norm(x):
    n = jnp.linalg.norm(x, axis=1, keepdims=True)
    return x / jnp.maximum(n, 1e-12)


def kernel(x, edge_index, W1, a1s, a1d, b1, W3, a3s, a3d, b3, p):
    loop = jnp.arange(N, dtype=jnp.int32)
    src = jnp.concatenate([edge_index[0], loop])
    dst = jnp.concatenate([edge_index[1], loop])
    pad_idx = N + (jnp.arange(EPG - EFULL, dtype=jnp.int32) % (NP - N))
    src_p = jnp.concatenate([src, pad_idx])
    dst_p = jnp.concatenate([dst, pad_idx])

    o1, _ = _gat_fast(x, W1, a1s, a1d, src_p, dst_p, dst, 16)
    h = _pallas_relu_b(o1, b1)
    h = _l2norm(h)
    o2, _ = _gat_fast(h, W3, a3s, a3d, src_p, dst_p, dst, 128)
    h2 = _l2norm(_pallas_addb(o2, b3))
    score = _pallas_score(h2, p)

    score_pad = _pad1(score, NP, -3e38)
    rank = _pallas_rank(score_pad)
    hf_p = jnp.pad(h2, ((0, NP - N), (0, 0)))
    return _sc_final(hf_p, score_pad, rank.reshape(80, 128))[:5000]


# speedup vs baseline: 3.7113x; 1.0007x over previous
import functools

import jax
import jax.numpy as jnp
import numpy as np
from jax import lax
from jax.experimental import pallas as pl
from jax.experimental.pallas import tpu as pltpu
from jax.experimental.pallas import tpu_sc as plsc

N = 10000
NP = 10240
EFULL = 330000
EPG = 360448
PER_W = EPG // 32
NCHUNK = PER_W // 128
NVREG = PER_W // 16

_MESH = plsc.VectorSubcoreMesh(core_axis_name="c", subcore_axis_name="s")
_SC_PARAMS = pltpu.CompilerParams(needs_layout_passes=False)


def _mm_kernel(x_ref, w_ref, o_ref):
    o_ref[...] = jnp.dot(x_ref[...], w_ref[...],
                         preferred_element_type=jnp.float32)


def _pallas_mm(x, W):
    Nr, K = x.shape
    _, F = W.shape
    BR = 1000
    return pl.pallas_call(
        _mm_kernel,
        grid=(Nr // BR,),
        in_specs=[pl.BlockSpec((BR, K), lambda i: (i, 0)),
                  pl.BlockSpec((K, F), lambda i: (0, 0))],
        out_specs=pl.BlockSpec((BR, F), lambda i: (i, 0)),
        out_shape=jax.ShapeDtypeStruct((Nr, F), jnp.float32),
    )(x, W)


def _pallas_proj(h, a):
    Nr, F = h.shape
    BR = 1000
    out = pl.pallas_call(
        _mm_kernel,
        grid=(Nr // BR,),
        in_specs=[pl.BlockSpec((BR, F), lambda i: (i, 0)),
                  pl.BlockSpec((F, 1), lambda i: (0, 0))],
        out_specs=pl.BlockSpec((BR, 1), lambda i: (i, 0)),
        out_shape=jax.ShapeDtypeStruct((Nr, 1), jnp.float32),
    )(h, a.reshape(F, 1))
    return out.reshape(Nr)


def _ee_kernel(e_ref, m_ref, o_ref):
    o_ref[...] = jnp.exp(e_ref[...] - m_ref[...])


def _alpha_kernel(ee_ref, d_ref, o_ref):
    o_ref[...] = ee_ref[...] / (d_ref[...] + 1e-16)


def _ew2d(kernel_fn, a, b):
    a2 = a.reshape(2816, 128)
    b2 = b.reshape(2816, 128)
    out = pl.pallas_call(
        kernel_fn,
        grid=(4,),
        in_specs=[pl.BlockSpec((704, 128), lambda i: (i, 0)),
                  pl.BlockSpec((704, 128), lambda i: (i, 0))],
        out_specs=pl.BlockSpec((704, 128), lambda i: (i, 0)),
        out_shape=jax.ShapeDtypeStruct((2816, 128), jnp.float32),
    )(a2, b2)
    return out.reshape(EPG)


def _make_upd_kernel(D):
    def _upd_kernel(g_ref, al_ref, o_ref):
        o_ref[...] = g_ref[:, :D] * al_ref[...]
    return _upd_kernel


def _pallas_upd(g, alpha, D):
    Er = g.shape[0]
    BR = 4096
    return pl.pallas_call(
        _make_upd_kernel(D),
        grid=(Er // BR,),
        in_specs=[pl.BlockSpec((BR, 128), lambda i: (i, 0)),
                  pl.BlockSpec((BR, 1), lambda i: (i, 0))],
        out_specs=pl.BlockSpec((BR, D), lambda i: (i, 0)),
        out_shape=jax.ShapeDtypeStruct((Er, D), jnp.float32),
    )(g, alpha.reshape(Er, 1))


def _relu_b_kernel(x_ref, b_ref, o_ref):
    o_ref[...] = jax.nn.relu(x_ref[...] + b_ref[...])


def _pallas_relu_b(x, b):
    Nr, D = x.shape
    BR = 1000
    return pl.pallas_call(
        _relu_b_kernel,
        grid=(Nr // BR,),
        in_specs=[pl.BlockSpec((BR, D), lambda i: (i, 0)),
                  pl.BlockSpec((1, D), lambda i: (0, 0))],
        out_specs=pl.BlockSpec((BR, D), lambda i: (i, 0)),
        out_shape=jax.ShapeDtypeStruct((Nr, D), jnp.float32),
    )(x, b.reshape(1, D))


def _addb_kernel(x_ref, b_ref, o_ref):
    o_ref[...] = x_ref[...] + b_ref[...]


def _pallas_addb(x, b):
    Nr, D = x.shape
    BR = 1000
    return pl.pallas_call(
        _addb_kernel,
        grid=(Nr // BR,),
        in_specs=[pl.BlockSpec((BR, D), lambda i: (i, 0)),
                  pl.BlockSpec((1, D), lambda i: (0, 0))],
        out_specs=pl.BlockSpec((BR, D), lambda i: (i, 0)),
        out_shape=jax.ShapeDtypeStruct((Nr, D), jnp.float32),
    )(x, b.reshape(1, D))


def _score_kernel(h_ref, p_ref, o_ref):
    p = p_ref[...]
    pn = jnp.sqrt(jnp.sum(p * p))
    u = jnp.dot(h_ref[...], p.reshape(-1, 1),
                preferred_element_type=jnp.float32)
    o_ref[...] = jnp.tanh(u / jnp.maximum(pn, 1e-12))


def _pallas_score(h, p):
    Nr, F = h.shape
    BR = 1000
    out = pl.pallas_call(
        _score_kernel,
        grid=(Nr // BR,),
        in_specs=[pl.BlockSpec((BR, F), lambda i: (i, 0)),
                  pl.BlockSpec((1, F), lambda i: (0, 0))],
        out_specs=pl.BlockSpec((BR, 1), lambda i: (i, 0)),
        out_shape=jax.ShapeDtypeStruct((Nr, 1), jnp.float32),
    )(h, p.reshape(1, F))
    return out.reshape(Nr)


def _rank_kernel(sc_ref, sr_ref, o_ref):
    pid = pl.program_id(0)
    si = sc_ref[...]
    gi = jax.lax.broadcasted_iota(jnp.int32, (1024, 1), 0) + pid * 1024
    acc = jnp.zeros((1024, 1), jnp.int32)
    for c in range(8):
        sj = sr_ref[:, c * 1280:(c + 1) * 1280]
        gj = jax.lax.broadcasted_iota(jnp.int32, (1, 1280), 1) + c * 1280
        hit = (sj > si) | ((sj == si) & (gj < gi))
        acc = acc + jnp.sum(hit.astype(jnp.int32), axis=1, keepdims=True)
    o_ref[...] = acc


def _pallas_rank(score_p):
    out = pl.pallas_call(
        _rank_kernel,
        grid=(NP // 1024,),
        in_specs=[pl.BlockSpec((1024, 1), lambda i: (i, 0)),
                  pl.BlockSpec((1, NP), lambda i: (0, 0))],
        out_specs=pl.BlockSpec((1024, 1), lambda i: (i, 0)),
        out_shape=jax.ShapeDtypeStruct((NP, 1), jnp.int32),
    )(score_p.reshape(NP, 1), score_p.reshape(1, NP))
    return out.reshape(NP)


@functools.partial(
    pl.kernel,
    out_type=jax.ShapeDtypeStruct((EPG,), jnp.float32),
    mesh=_MESH,
    compiler_params=_SC_PARAMS,
    scratch_types=[
        pltpu.VMEM((NP,), jnp.float32),
        pltpu.VMEM((NP,), jnp.float32),
        pltpu.VMEM((PER_W,), jnp.int32),
        pltpu.VMEM((PER_W,), jnp.int32),
        pltpu.VMEM((PER_W,), jnp.float32),
    ],
)
def _sc_edge_logits(src_hbm, dst_hbm, as_hbm, ad_hbm, e_hbm,
                    as_v, ad_v, sv, dv, ev):
    wid = lax.axis_index("c") * 16 + lax.axis_index("s")
    base = wid * PER_W
    pltpu.sync_copy(as_hbm, as_v)
    pltpu.sync_copy(ad_hbm, ad_v)
    pltpu.sync_copy(src_hbm.at[pl.ds(base, PER_W)], sv)
    pltpu.sync_copy(dst_hbm.at[pl.ds(base, PER_W)], dv)

    def body(i, _):
        for u in range(8):
            off = (i * 8 + u) * 16
            s16 = sv[pl.ds(off, 16)]
            d16 = dv[pl.ds(off, 16)]
            a = plsc.load_gather(as_v, [s16])
            bvals = plsc.load_gather(ad_v, [d16])
            v = a + bvals
            ev[pl.ds(off, 16)] = jnp.where(v >= 0, v, jnp.float32(0.2) * v)
        return ()
    lax.fori_loop(0, NVREG // 8, body, ())
    pltpu.sync_copy(ev, e_hbm.at[pl.ds(base, PER_W)])


@functools.partial(
    pl.kernel,
    out_type=jax.ShapeDtypeStruct((EPG,), jnp.float32),
    mesh=_MESH,
    compiler_params=_SC_PARAMS,
    scratch_types=[
        pltpu.VMEM((NP,), jnp.float32),
        pltpu.VMEM((PER_W,), jnp.int32),
        pltpu.VMEM((PER_W,), jnp.float32),
    ],
)
def _sc_gather1(table_hbm, idx_hbm, o_hbm, tv, iv, ov):
    wid = lax.axis_index("c") * 16 + lax.axis_index("s")
    base = wid * PER_W
    pltpu.sync_copy(table_hbm, tv)
    pltpu.sync_copy(idx_hbm.at[pl.ds(base, PER_W)], iv)

    def body(i, _):
        for u in range(8):
            off = (i * 8 + u) * 16
            i16 = iv[pl.ds(off, 16)]
            ov[pl.ds(off, 16)] = plsc.load_gather(tv, [i16])
        return ()
    lax.fori_loop(0, NVREG // 8, body, ())
    pltpu.sync_copy(ov, o_hbm.at[pl.ds(base, PER_W)])


@functools.partial(
    pl.kernel,
    out_type=jax.ShapeDtypeStruct((EPG, 128), jnp.float32),
    mesh=_MESH,
    compiler_params=_SC_PARAMS,
    scratch_types=[
        pltpu.VMEM((PER_W,), jnp.int32),
        pltpu.VMEM((128, 128), jnp.float32),
        pltpu.VMEM((128, 128), jnp.float32),
        pltpu.SemaphoreType.DMA,
        pltpu.SemaphoreType.DMA,
    ],
)
def _sc_rowgather(h_hbm, src_hbm, o_hbm, iv, rows_a, rows_b, sem_a, sem_b):
    wid = lax.axis_index("c") * 16 + lax.axis_index("s")
    base = wid * PER_W
    pltpu.sync_copy(src_hbm.at[pl.ds(base, PER_W)], iv)

    pltpu.async_copy(h_hbm.at[iv.at[pl.ds(0, 128)]], rows_a, sem_a)

    def body(j, _):
        c0 = 2 * j
        c1 = 2 * j + 1
        pltpu.async_copy(
            h_hbm.at[iv.at[pl.ds(c1 * 128, 128)]], rows_b, sem_b)
        pltpu.make_async_copy(
            h_hbm.at[iv.at[pl.ds(0, 128)]], rows_a, sem_a).wait()
        pltpu.sync_copy(rows_a, o_hbm.at[pl.ds(base + c0 * 128, 128), :])

        @pl.when(j + 1 < NCHUNK // 2)
        def _():
            pltpu.async_copy(
                h_hbm.at[iv.at[pl.ds((c0 + 2) * 128, 128)]], rows_a, sem_a)
        pltpu.make_async_copy(
            h_hbm.at[iv.at[pl.ds(0, 128)]], rows_b, sem_b).wait()
        pltpu.sync_copy(rows_b, o_hbm.at[pl.ds(base + c1 * 128, 128), :])
        return ()
    lax.fori_loop(0, NCHUNK // 2, body, ())


@functools.partial(
    pl.kernel,
    out_type=jax.ShapeDtypeStruct((NP, 128), jnp.float32),
    mesh=_MESH,
    compiler_params=_SC_PARAMS,
    scratch_types=[
        pltpu.VMEM((NP,), jnp.float32),
        pltpu.VMEM((80, 128), jnp.int32),
        pltpu.VMEM((640, 128), jnp.float32),
        pltpu.SemaphoreType.DMA,
    ],
)
def _sc_final(hf_hbm, score_hbm, rank2d_hbm, o_hbm, sv, rv2, rows_v, sem):
    wid = lax.axis_index("c") * 16 + lax.axis_index("s")

    @pl.when(wid < 16)
    def _():
        base = wid * 640
        pltpu.sync_copy(score_hbm, sv)
        pltpu.sync_copy(rank2d_hbm, rv2)
        pltpu.sync_copy(hf_hbm.at[pl.ds(base, 640), :], rows_v)

        def scale_body(r, _):
            s16 = plsc.load_gather(sv, [jnp.full((16,), base + r, jnp.int32)])
            for k in range(8):
                rows_v[r, pl.ds(k * 16, 16)] = (
                    rows_v[r, pl.ds(k * 16, 16)] * s16)
            return ()
        lax.fori_loop(0, 640, scale_body, ())

        def scat_body(j, _):
            pltpu.async_copy(
                rows_v.at[pl.ds(j * 128, 128), :],
                o_hbm.at[rv2.at[wid * 5 + j]], sem).wait()
            return ()
        lax.fori_loop(0, 5, scat_body, ())


def _pad1(v, n, val=0.0):
    return jnp.pad(v, (0, n - v.shape[0]), constant_values=val)


def _gat_fast(x, W, a_s, a_d, src_p, dst_p, dst, D):
    Nn = x.shape[0]
    h = _pallas_mm(x, W)
    alpha_src = _pallas_proj(h, a_s)
    alpha_dst = _pallas_proj(h, a_d)
    e = _sc_edge_logits(src_p, dst_p, _pad1(alpha_src, NP),
                        _pad1(alpha_dst, NP))
    emax = jax.ops.segment_max(e[:EFULL], dst, num_segments=Nn)
    emax = jax.lax.stop_gradient(emax)
    emd = _sc_gather1(_pad1(emax, NP), dst_p)
    ee = _ew2d(_ee_kernel, e, emd)
    den = jax.ops.segment_sum(ee[:EFULL], dst, num_segments=Nn)
    dend = _sc_gather1(_pad1(den, NP), dst_p)
    alpha = _ew2d(_alpha_kernel, ee, dend)
    h_p = jnp.pad(h, ((0, NP - Nn), (0, 128 - D)))
    g = _sc_rowgather(h_p, src_p)
    upd = _pallas_upd(g, alpha, D)
    out = jax.ops.segment_sum(upd[:EFULL], dst, num_segments=Nn)
    return out, h


def _l2norm(x):
    n = jnp.linalg.norm(x, axis=1, keepdims=True)
    return x / jnp.maximum(n, 1e-12)


def kernel(x, edge_index, W1, a1s, a1d, b1, W3, a3s, a3d, b3, p):
    loop = jnp.arange(N, dtype=jnp.int32)
    src = jnp.concatenate([edge_index[0], loop])
    dst = jnp.concatenate([edge_index[1], loop])
    pad_idx = N + (jnp.arange(EPG - EFULL, dtype=jnp.int32) % (NP - N))
    src_p = jnp.concatenate([src, pad_idx])
    dst_p = jnp.concatenate([dst, pad_idx])

    o1, _ = _gat_fast(x, W1, a1s, a1d, src_p, dst_p, dst, 16)
    h = _pallas_relu_b(o1, b1)
    h = _l2norm(h)
    o2, _ = _gat_fast(h, W3, a3s, a3d, src_p, dst_p, dst, 128)
    h2 = _l2norm(_pallas_addb(o2, b3))
    score = _pallas_score(h2, p)

    score_pad = _pad1(score, NP, -3e38)
    rank = _pallas_rank(score_pad)
    hf_p = jnp.pad(h2, ((0, NP - N), (0, 0)))
    return _sc_final(hf_p, score_pad, rank.reshape(80, 128))[:5000]


# speedup vs baseline: 3.8698x; 1.0427x over previous
import functools

import jax
import jax.numpy as jnp
import numpy as np
from jax import lax
from jax.experimental import pallas as pl
from jax.experimental.pallas import tpu as pltpu
from jax.experimental.pallas import tpu_sc as plsc

N = 10000
NP = 10240
EFULL = 330000
EPG = 360448
PER_W = EPG // 32
NCHUNK = PER_W // 128
NVREG = PER_W // 16

_MESH = plsc.VectorSubcoreMesh(core_axis_name="c", subcore_axis_name="s")
_SC_PARAMS = pltpu.CompilerParams(needs_layout_passes=False)


def _mm_kernel(x_ref, w_ref, o_ref):
    o_ref[...] = jnp.dot(x_ref[...], w_ref[...],
                         preferred_element_type=jnp.float32)


def _pallas_mm(x, W):
    Nr, K = x.shape
    _, F = W.shape
    BR = 1000
    return pl.pallas_call(
        _mm_kernel,
        grid=(Nr // BR,),
        in_specs=[pl.BlockSpec((BR, K), lambda i: (i, 0)),
                  pl.BlockSpec((K, F), lambda i: (0, 0))],
        out_specs=pl.BlockSpec((BR, F), lambda i: (i, 0)),
        out_shape=jax.ShapeDtypeStruct((Nr, F), jnp.float32),
    )(x, W)


def _pallas_proj(h, a):
    Nr, F = h.shape
    BR = 1000
    out = pl.pallas_call(
        _mm_kernel,
        grid=(Nr // BR,),
        in_specs=[pl.BlockSpec((BR, F), lambda i: (i, 0)),
                  pl.BlockSpec((F, 1), lambda i: (0, 0))],
        out_specs=pl.BlockSpec((BR, 1), lambda i: (i, 0)),
        out_shape=jax.ShapeDtypeStruct((Nr, 1), jnp.float32),
    )(h, a.reshape(F, 1))
    return out.reshape(Nr)


def _ee_kernel(e_ref, m_ref, o_ref):
    o_ref[...] = jnp.exp(e_ref[...] - m_ref[...])


def _alpha_kernel(ee_ref, d_ref, o_ref):
    o_ref[...] = ee_ref[...] / (d_ref[...] + 1e-16)


def _ew2d(kernel_fn, a, b):
    a2 = a.reshape(2816, 128)
    b2 = b.reshape(2816, 128)
    out = pl.pallas_call(
        kernel_fn,
        grid=(4,),
        in_specs=[pl.BlockSpec((704, 128), lambda i: (i, 0)),
                  pl.BlockSpec((704, 128), lambda i: (i, 0))],
        out_specs=pl.BlockSpec((704, 128), lambda i: (i, 0)),
        out_shape=jax.ShapeDtypeStruct((2816, 128), jnp.float32),
    )(a2, b2)
    return out.reshape(EPG)


def _make_upd_kernel(D):
    def _upd_kernel(g_ref, al_ref, o_ref):
        o_ref[...] = g_ref[:, :D] * al_ref[...]
    return _upd_kernel


def _pallas_upd(g, alpha, D):
    Er = g.shape[0]
    BR = 4096
    nblk = (EFULL + BR - 1) // BR
    return pl.pallas_call(
        _make_upd_kernel(D),
        grid=(nblk,),
        in_specs=[pl.BlockSpec((BR, 128), lambda i: (i, 0)),
                  pl.BlockSpec((BR, 1), lambda i: (i, 0))],
        out_specs=pl.BlockSpec((BR, D), lambda i: (i, 0)),
        out_shape=jax.ShapeDtypeStruct((EFULL, D), jnp.float32),
    )(g, alpha.reshape(Er, 1))


def _relu_b_kernel(x_ref, b_ref, o_ref):
    o_ref[...] = jax.nn.relu(x_ref[...] + b_ref[...])


def _pallas_relu_b(x, b):
    Nr, D = x.shape
    BR = 1000
    return pl.pallas_call(
        _relu_b_kernel,
        grid=(Nr // BR,),
        in_specs=[pl.BlockSpec((BR, D), lambda i: (i, 0)),
                  pl.BlockSpec((1, D), lambda i: (0, 0))],
        out_specs=pl.BlockSpec((BR, D), lambda i: (i, 0)),
        out_shape=jax.ShapeDtypeStruct((Nr, D), jnp.float32),
    )(x, b.reshape(1, D))


def _addb_kernel(x_ref, b_ref, o_ref):
    o_ref[...] = x_ref[...] + b_ref[...]


def _pallas_addb(x, b):
    Nr, D = x.shape
    BR = 1000
    return pl.pallas_call(
        _addb_kernel,
        grid=(Nr // BR,),
        in_specs=[pl.BlockSpec((BR, D), lambda i: (i, 0)),
                  pl.BlockSpec((1, D), lambda i: (0, 0))],
        out_specs=pl.BlockSpec((BR, D), lambda i: (i, 0)),
        out_shape=jax.ShapeDtypeStruct((Nr, D), jnp.float32),
    )(x, b.reshape(1, D))


def _score_kernel(h_ref, p_ref, o_ref):
    p = p_ref[...]
    pn = jnp.sqrt(jnp.sum(p * p))
    u = jnp.dot(h_ref[...], p.reshape(-1, 1),
                preferred_element_type=jnp.float32)
    o_ref[...] = jnp.tanh(u / jnp.maximum(pn, 1e-12))


def _pallas_score(h, p):
    Nr, F = h.shape
    BR = 1000
    out = pl.pallas_call(
        _score_kernel,
        grid=(Nr // BR,),
        in_specs=[pl.BlockSpec((BR, F), lambda i: (i, 0)),
                  pl.BlockSpec((1, F), lambda i: (0, 0))],
        out_specs=pl.BlockSpec((BR, 1), lambda i: (i, 0)),
        out_shape=jax.ShapeDtypeStruct((Nr, 1), jnp.float32),
    )(h, p.reshape(1, F))
    return out.reshape(Nr)


def _rank_kernel(sc_ref, sr_ref, o_ref):
    pid = pl.program_id(0)
    si = sc_ref[...]
    gi = jax.lax.broadcasted_iota(jnp.int32, (1024, 1), 0) + pid * 1024
    acc = jnp.zeros((1024, 1), jnp.int32)
    for c in range(8):
        sj = sr_ref[:, c * 1280:(c + 1) * 1280]
        gj = jax.lax.broadcasted_iota(jnp.int32, (1, 1280), 1) + c * 1280
        hit = (sj > si) | ((sj == si) & (gj < gi))
        acc = acc + jnp.sum(hit.astype(jnp.int32), axis=1, keepdims=True)
    o_ref[...] = acc


def _pallas_rank(score_p):
    out = pl.pallas_call(
        _rank_kernel,
        grid=(NP // 1024,),
        in_specs=[pl.BlockSpec((1024, 1), lambda i: (i, 0)),
                  pl.BlockSpec((1, NP), lambda i: (0, 0))],
        out_specs=pl.BlockSpec((1024, 1), lambda i: (i, 0)),
        out_shape=jax.ShapeDtypeStruct((NP, 1), jnp.int32),
    )(score_p.reshape(NP, 1), score_p.reshape(1, NP))
    return out.reshape(NP)


@functools.partial(
    pl.kernel,
    out_type=jax.ShapeDtypeStruct((EPG,), jnp.float32),
    mesh=_MESH,
    compiler_params=_SC_PARAMS,
    scratch_types=[
        pltpu.VMEM((NP,), jnp.float32),
        pltpu.VMEM((NP,), jnp.float32),
        pltpu.VMEM((PER_W,), jnp.int32),
        pltpu.VMEM((PER_W,), jnp.int32),
        pltpu.VMEM((PER_W,), jnp.float32),
    ],
)
def _sc_edge_logits(src_hbm, dst_hbm, as_hbm, ad_hbm, e_hbm,
                    as_v, ad_v, sv, dv, ev):
    wid = lax.axis_index("c") * 16 + lax.axis_index("s")
    base = wid * PER_W
    pltpu.sync_copy(as_hbm, as_v)
    pltpu.sync_copy(ad_hbm, ad_v)
    pltpu.sync_copy(src_hbm.at[pl.ds(base, PER_W)], sv)
    pltpu.sync_copy(dst_hbm.at[pl.ds(base, PER_W)], dv)

    def body(i, _):
        for u in range(8):
            off = (i * 8 + u) * 16
            s16 = sv[pl.ds(off, 16)]
            d16 = dv[pl.ds(off, 16)]
            a = plsc.load_gather(as_v, [s16])
            bvals = plsc.load_gather(ad_v, [d16])
            v = a + bvals
            ev[pl.ds(off, 16)] = jnp.where(v >= 0, v, jnp.float32(0.2) * v)
        return ()
    lax.fori_loop(0, NVREG // 8, body, ())
    pltpu.sync_copy(ev, e_hbm.at[pl.ds(base, PER_W)])


@functools.partial(
    pl.kernel,
    out_type=jax.ShapeDtypeStruct((EPG,), jnp.float32),
    mesh=_MESH,
    compiler_params=_SC_PARAMS,
    scratch_types=[
        pltpu.VMEM((NP,), jnp.float32),
        pltpu.VMEM((PER_W,), jnp.int32),
        pltpu.VMEM((PER_W,), jnp.float32),
    ],
)
def _sc_gather1(table_hbm, idx_hbm, o_hbm, tv, iv, ov):
    wid = lax.axis_index("c") * 16 + lax.axis_index("s")
    base = wid * PER_W
    pltpu.sync_copy(table_hbm, tv)
    pltpu.sync_copy(idx_hbm.at[pl.ds(base, PER_W)], iv)

    def body(i, _):
        for u in range(8):
            off = (i * 8 + u) * 16
            i16 = iv[pl.ds(off, 16)]
            ov[pl.ds(off, 16)] = plsc.load_gather(tv, [i16])
        return ()
    lax.fori_loop(0, NVREG // 8, body, ())
    pltpu.sync_copy(ov, o_hbm.at[pl.ds(base, PER_W)])


@functools.partial(
    pl.kernel,
    out_type=jax.ShapeDtypeStruct((EPG, 128), jnp.float32),
    mesh=_MESH,
    compiler_params=_SC_PARAMS,
    scratch_types=[
        pltpu.VMEM((PER_W,), jnp.int32),
        pltpu.VMEM((128, 128), jnp.float32),
        pltpu.VMEM((128, 128), jnp.float32),
        pltpu.SemaphoreType.DMA,
        pltpu.SemaphoreType.DMA,
    ],
)
def _sc_rowgather(h_hbm, src_hbm, o_hbm, iv, rows_a, rows_b, sem_a, sem_b):
    wid = lax.axis_index("c") * 16 + lax.axis_index("s")
    base = wid * PER_W
    pltpu.sync_copy(src_hbm.at[pl.ds(base, PER_W)], iv)

    pltpu.async_copy(h_hbm.at[iv.at[pl.ds(0, 128)]], rows_a, sem_a)

    def body(j, _):
        c0 = 2 * j
        c1 = 2 * j + 1
        pltpu.async_copy(
            h_hbm.at[iv.at[pl.ds(c1 * 128, 128)]], rows_b, sem_b)
        pltpu.make_async_copy(
            h_hbm.at[iv.at[pl.ds(0, 128)]], rows_a, sem_a).wait()
        pltpu.sync_copy(rows_a, o_hbm.at[pl.ds(base + c0 * 128, 128), :])

        @pl.when(j + 1 < NCHUNK // 2)
        def _():
            pltpu.async_copy(
                h_hbm.at[iv.at[pl.ds((c0 + 2) * 128, 128)]], rows_a, sem_a)
        pltpu.make_async_copy(
            h_hbm.at[iv.at[pl.ds(0, 128)]], rows_b, sem_b).wait()
        pltpu.sync_copy(rows_b, o_hbm.at[pl.ds(base + c1 * 128, 128), :])
        return ()
    lax.fori_loop(0, NCHUNK // 2, body, ())


@functools.partial(
    pl.kernel,
    out_type=jax.ShapeDtypeStruct((NP, 128), jnp.float32),
    mesh=_MESH,
    compiler_params=_SC_PARAMS,
    scratch_types=[
        pltpu.VMEM((NP,), jnp.float32),
        pltpu.VMEM((80, 128), jnp.int32),
        pltpu.VMEM((640, 128), jnp.float32),
        pltpu.SemaphoreType.DMA,
    ],
)
def _sc_final(hf_hbm, score_hbm, rank2d_hbm, o_hbm, sv, rv2, rows_v, sem):
    wid = lax.axis_index("c") * 16 + lax.axis_index("s")

    @pl.when(wid < 16)
    def _():
        base = wid * 640
        pltpu.sync_copy(score_hbm, sv)
        pltpu.sync_copy(rank2d_hbm, rv2)
        pltpu.sync_copy(hf_hbm.at[pl.ds(base, 640), :], rows_v)

        def scale_body(r, _):
            s16 = plsc.load_gather(sv, [jnp.full((16,), base + r, jnp.int32)])
            for k in range(8):
                rows_v[r, pl.ds(k * 16, 16)] = (
                    rows_v[r, pl.ds(k * 16, 16)] * s16)
            return ()
        lax.fori_loop(0, 640, scale_body, ())

        def scat_body(j, _):
            pltpu.async_copy(
                rows_v.at[pl.ds(j * 128, 128), :],
                o_hbm.at[rv2.at[wid * 5 + j]], sem).wait()
            return ()
        lax.fori_loop(0, 5, scat_body, ())


def _pad1(v, n, val=0.0):
    return jnp.pad(v, (0, n - v.shape[0]), constant_values=val)


def _gat_fast(x, W, a_s, a_d, src_p, dst_p, dst, D):
    Nn = x.shape[0]
    h = _pallas_mm(x, W)
    alpha_src = _pallas_proj(h, a_s)
    alpha_dst = _pallas_proj(h, a_d)
    e = _sc_edge_logits(src_p, dst_p, _pad1(alpha_src, NP),
                        _pad1(alpha_dst, NP))
    emax = jax.ops.segment_max(e[:EFULL], dst, num_segments=Nn)
    emax = jax.lax.stop_gradient(emax)
    emd = _sc_gather1(_pad1(emax, NP), dst_p)
    ee = _ew2d(_ee_kernel, e, emd)
    den = jax.ops.segment_sum(ee[:EFULL], dst, num_segments=Nn)
    dend = _sc_gather1(_pad1(den, NP), dst_p)
    alpha = _ew2d(_alpha_kernel, ee, dend)
    h_p = jnp.pad(h, ((0, NP - Nn), (0, 128 - D)))
    g = _sc_rowgather(h_p, src_p)
    upd = _pallas_upd(g, alpha, D)
    out = jax.ops.segment_sum(upd, dst, num_segments=Nn)
    return out, h


def _l2norm(x):
    n = jnp.linalg.norm(x, axis=1, keepdims=True)
    return x / jnp.maximum(n, 1e-12)


def kernel(x, edge_index, W1, a1s, a1d, b1, W3, a3s, a3d, b3, p):
    loop = jnp.arange(N, dtype=jnp.int32)
    src = jnp.concatenate([edge_index[0], loop])
    dst = jnp.concatenate([edge_index[1], loop])
    pad_idx = N + (jnp.arange(EPG - EFULL, dtype=jnp.int32) % (NP - N))
    src_p = jnp.concatenate([src, pad_idx])
    dst_p = jnp.concatenate([dst, pad_idx])

    o1, _ = _gat_fast(x, W1, a1s, a1d, src_p, dst_p, dst, 16)
    h = _pallas_relu_b(o1, b1)
    h = _l2norm(h)
    o2, _ = _gat_fast(h, W3, a3s, a3d, src_p, dst_p, dst, 128)
    h2 = _l2norm(_pallas_addb(o2, b3))
    score = _pallas_score(h2, p)

    score_pad = _pad1(score, NP, -3e38)
    rank = _pallas_rank(score_pad)
    hf_p = jnp.pad(h2, ((0, NP - N), (0, 0)))
    return _sc_final(hf_p, score_pad, rank.reshape(80, 128))[:5000]
